# baseline (speedup 1.0000x reference)
import functools

import jax
import jax.numpy as jnp
from jax import lax
from jax.experimental import pallas as pl
from jax.experimental.pallas import tpu as pltpu


def _fused_body(adj_ref, x_ref, w_ref, b_ref, out_ref):
    h = jnp.dot(adj_ref[...], x_ref[...], preferred_element_type=jnp.float32)
    y = lax.dot_general(
        h, w_ref[...], (((1,), (1,)), ((), ())),
        preferred_element_type=jnp.float32,
    )
    out_ref[...] = jnp.maximum(y + b_ref[...], 0.0)


@functools.partial(jax.jit, static_argnames=("block_m",))
def _fused_graph_layer(x, adj, W, b2d, block_m):
    n, d_in = x.shape
    d_out = W.shape[0]
    grid = (pl.cdiv(n, block_m),)
    return pl.pallas_call(
        _fused_body,
        grid=grid,
        in_specs=[
            pl.BlockSpec((block_m, n), lambda i: (i, 0)),
            pl.BlockSpec((n, d_in), lambda i: (0, 0)),
            pl.BlockSpec((d_out, d_in), lambda i: (0, 0)),
            pl.BlockSpec((1, d_out), lambda i: (0, 0)),
        ],
        out_specs=pl.BlockSpec((block_m, d_out), lambda i: (i, 0)),
        out_shape=jax.ShapeDtypeStruct((n, d_out), jnp.float32),
    )(adj, x, W, b2d)


def kernel(x, adj, W, b):
    b2d = b.reshape(1, -1)
    return _fused_graph_layer(x, adj, W, b2d, block_m=640)


# speedup vs baseline: 1.0259x; 1.0259x over previous
import functools

import jax
import jax.numpy as jnp
from jax import lax
from jax.experimental import pallas as pl
from jax.experimental.pallas import tpu as pltpu


def _fused_body(adj_ref, x_ref, w_ref, b_ref, out_ref):
    h = jnp.dot(adj_ref[...], x_ref[...], preferred_element_type=jnp.float32)
    y = lax.dot_general(
        h, w_ref[...], (((1,), (1,)), ((), ())),
        preferred_element_type=jnp.float32,
    )
    out_ref[...] = jnp.maximum(y + b_ref[...], 0.0)


@functools.partial(jax.jit, static_argnames=("block_m",))
def _fused_graph_layer(x, adj, W, b2d, block_m):
    n, d_in = x.shape
    d_out = W.shape[0]
    grid = (pl.cdiv(n, block_m),)
    return pl.pallas_call(
        _fused_body,
        grid=grid,
        in_specs=[
            pl.BlockSpec((block_m, n), lambda i: (i, 0)),
            pl.BlockSpec((n, d_in), lambda i: (0, 0)),
            pl.BlockSpec((d_out, d_in), lambda i: (0, 0)),
            pl.BlockSpec((1, d_out), lambda i: (0, 0)),
        ],
        out_specs=pl.BlockSpec((block_m, d_out), lambda i: (i, 0)),
        out_shape=jax.ShapeDtypeStruct((n, d_out), jnp.float32),
    )(adj, x, W, b2d)


def kernel(x, adj, W, b):
    b2d = b.reshape(1, -1)
    return _fused_graph_layer(x, adj, W, b2d, block_m=400)
